# baseline (speedup 1.0000x reference)
import functools

import jax
import jax.numpy as jnp
from jax import lax
from jax.experimental import pallas as pl
from jax.experimental.pallas import tpu as pltpu
from jax.experimental.pallas import tpu_sc as plsc

_NG = 64
_K = 128
_CW = 128
_NSC = 2
_NTILE = 16


def _blk(total):
    for b in (640, 512, 256, 128):
        if total % b == 0:
            return b
    return total


def _silu(v):
    return v * jax.nn.sigmoid(v)


def _split_w(w):
    hi = w.astype(jnp.bfloat16)
    lo = (w - hi.astype(jnp.float32)).astype(jnp.bfloat16)
    return hi, lo


def _dot3(v, whi, wlo):
    vhi = v.astype(jnp.bfloat16)
    vlo = (v - vhi.astype(jnp.float32)).astype(jnp.bfloat16)
    return (jnp.dot(vhi, whi, preferred_element_type=jnp.float32)
            + jnp.dot(vhi, wlo, preferred_element_type=jnp.float32)
            + jnp.dot(vlo, whi, preferred_element_type=jnp.float32))


@functools.lru_cache(maxsize=None)
def _make_gather(ep, fp, np_):
    nworkers = _NSC * _NTILE
    epw = ep // nworkers
    nwin = epw // _K
    mesh = plsc.VectorSubcoreMesh(
        core_axis_name="c", subcore_axis_name="s",
        num_cores=_NSC, num_subcores=_NTILE)

    def body(ts, td, src, dst, gs, gd, idx_v, rows_v, sem):
        wid = lax.axis_index("s") * _NSC + lax.axis_index("c")
        base0 = wid * epw

        def run(table, idx_hbm, out_hbm):
            def w_body(w, carry):
                base = base0 + w * _K
                pltpu.sync_copy(idx_hbm.at[pl.ds(base, _K)], idx_v)
                pltpu.async_copy(table.at[idx_v], rows_v, sem).wait()
                pltpu.sync_copy(rows_v, out_hbm.at[pl.ds(base, _K)])
                return carry
            lax.fori_loop(0, nwin, w_body, 0)

        run(ts, src, gs)
        run(td, dst, gd)

    return pl.kernel(
        body,
        out_type=(jax.ShapeDtypeStruct((ep, fp), jnp.float32),
                  jax.ShapeDtypeStruct((ep, fp), jnp.float32)),
        mesh=mesh,
        scratch_types=[pltpu.VMEM((_K,), jnp.int32),
                       pltpu.VMEM((_K, fp), jnp.float32),
                       pltpu.SemaphoreType.DMA],
    )


@functools.lru_cache(maxsize=None)
def _make_scatter(ep, np_, nch):
    ept = ep // _NTILE
    nwin = ept // _K
    rpt = np_ // _NTILE
    n0 = (nch + 1) // 2
    mesh = plsc.VectorSubcoreMesh(
        core_axis_name="c", subcore_axis_name="s",
        num_cores=_NSC, num_subcores=_NTILE)

    def body(dst, zrows, *refs):
        upd = refs[:nch]
        out = refs[nch:2 * nch]
        acc, idx_v, upd_v, sem = refs[2 * nch:]
        c = lax.axis_index("c")
        s = lax.axis_index("s")

        def process(u_hbm, o_hbm):
            pltpu.sync_copy(zrows, acc.at[pl.ds(s * rpt, rpt)])
            plsc.subcore_barrier()

            def w_body(w, carry):
                base = s * ept + w * _K
                pltpu.sync_copy(dst.at[pl.ds(base, _K)], idx_v)
                pltpu.sync_copy(u_hbm.at[pl.ds(base, _K)], upd_v)
                pltpu.sync_copy(upd_v, acc.at[idx_v], add=True)
                return carry
            lax.fori_loop(0, nwin, w_body, 0)
            plsc.subcore_barrier()
            pltpu.sync_copy(acc.at[pl.ds(s * rpt, rpt)],
                            o_hbm.at[pl.ds(s * rpt, rpt)])

        for j in range(n0):
            @pl.when(c == 0)
            def _(j=j):
                process(upd[j], out[j])
        for j in range(n0, nch):
            @pl.when(c == 1)
            def _(j=j):
                process(upd[j], out[j])

    return pl.kernel(
        body,
        out_type=tuple(jax.ShapeDtypeStruct((np_, _CW), jnp.float32)
                       for _ in range(nch)),
        mesh=mesh,
        scratch_types=[pltpu.VMEM_SHARED((np_, _CW), jnp.float32),
                       pltpu.VMEM((_K,), jnp.int32),
                       pltpu.VMEM((_K, _CW), jnp.float32),
                       pltpu.SemaphoreType.DMA],
    )


def _project(table, wshi, wslo, wdhi, wdlo):
    np_, fp = table.shape
    hid = wshi.shape[1]
    fpo = _ceil_to(hid + 3, 128)
    _BN = _blk(np_)
    grid = np_ // _BN

    def kern(t_r, wshi_r, wslo_r, wdhi_r, wdlo_r, ts_r, td_r):
        tv = t_r[...]
        coord = tv[:, 0:3]
        thi = tv.astype(jnp.bfloat16)
        tlo = (tv - thi.astype(jnp.float32)).astype(jnp.bfloat16)
        zpad = jnp.zeros((_BN, fpo - hid - 3), jnp.float32)
        for whi_r, wlo_r, o_r in ((wshi_r, wslo_r, ts_r),
                                  (wdhi_r, wdlo_r, td_r)):
            z = (jnp.dot(thi, whi_r[...], preferred_element_type=jnp.float32)
                 + jnp.dot(thi, wlo_r[...], preferred_element_type=jnp.float32)
                 + jnp.dot(tlo, whi_r[...], preferred_element_type=jnp.float32))
            o_r[...] = jnp.concatenate([z, coord, zpad], axis=1)

    full = lambda a: pl.BlockSpec(a.shape, lambda i: (0,) * a.ndim)
    return pl.pallas_call(
        kern,
        grid=(grid,),
        in_specs=[pl.BlockSpec((_BN, fp), lambda i: (i, 0)),
                  full(wshi), full(wslo), full(wdhi), full(wdlo)],
        out_specs=[pl.BlockSpec((_BN, fpo), lambda i: (i, 0))] * 2,
        out_shape=[jax.ShapeDtypeStruct((np_, fpo), jnp.float32)] * 2,
    )(table, wshi, wslo, wdhi, wdlo)


def _edge_mlp(gs, gd, ea, wr, we, b1, w2hi, w2lo, b2, cw1, cb1, cw2r,
              n_edges, nch):
    ep, fp = gs.shape
    hid = w2hi.shape[1]
    ef = ea.shape[1]
    _BE = _blk(ep)
    grid = ep // _BE

    def kern(gs_r, gd_r, ea_r, wr_r, we_r, b1_r, w2hi_r, w2lo_r, b2_r,
             cw1_r, cb1_r, cw2r_r, *outs):
        i = pl.program_id(0)
        gsv = gs_r[...]
        gdv = gd_r[...]
        xd = gsv[:, hid:hid + 3] - gdv[:, hid:hid + 3]
        radial = jnp.sum(xd * xd, axis=1, keepdims=True)
        xdn = xd / (jnp.sqrt(radial) + 1e-30)
        z1 = (gsv[:, :hid] + gdv[:, :hid]
              + jnp.dot(ea_r[...], we_r[...],
                        preferred_element_type=jnp.float32)
              + radial * wr_r[...] + b1_r[...])
        m1 = _silu(z1)
        m = _silu(_dot3(m1, w2hi_r[...], w2lo_r[...]) + b2_r[...])
        u = _silu(jnp.dot(m, cw1_r[...], preferred_element_type=jnp.float32)
                  + cb1_r[...])
        c = jnp.sum(u * cw2r_r[...], axis=1, keepdims=True)
        eid = i * _BE + lax.broadcasted_iota(jnp.int32, (_BE, 1), 0)
        mask = (eid < n_edges).astype(jnp.float32)
        mm = m * mask
        msgx = c * xdn * mask
        for j in range(nch - 1):
            outs[j][...] = mm[:, j * _CW:(j + 1) * _CW]
        pieces = []
        if hid > (nch - 1) * _CW:
            pieces.append(mm[:, (nch - 1) * _CW:])
        pieces += [msgx, mask]
        if nch * _CW - hid - 4 > 0:
            pieces.append(jnp.zeros((_BE, nch * _CW - hid - 4), jnp.float32))
        outs[nch - 1][...] = jnp.concatenate(pieces, axis=1)

    full = lambda a: pl.BlockSpec(a.shape, lambda i: (0,) * a.ndim)
    return pl.pallas_call(
        kern,
        grid=(grid,),
        in_specs=[
            pl.BlockSpec((_BE, fp), lambda i: (i, 0)),
            pl.BlockSpec((_BE, fp), lambda i: (i, 0)),
            pl.BlockSpec((_BE, ef), lambda i: (i, 0)),
            full(wr), full(we), full(b1),
            full(w2hi), full(w2lo), full(b2), full(cw1), full(cb1),
            full(cw2r),
        ],
        out_specs=[pl.BlockSpec((_BE, _CW), lambda i: (i, 0))
                   for _ in range(nch)],
        out_shape=[jax.ShapeDtypeStruct((ep, _CW), jnp.float32)
                   for _ in range(nch)],
    )(gs, gd, ea, wr, we, b1, w2hi, w2lo, b2, cw1, cb1, cw2r)


def _node_update(table, chunks, wh, wn, nb1, nw2, nb2, nch):
    np_, fp = table.shape
    f = wh.shape[0]
    hid = wn.shape[0]
    out_f = nw2.shape[1]
    fp_out = out_f + 16
    _BN = _blk(np_)
    grid = np_ // _BN
    whp = _split_w(wh)
    wnp = _split_w(wn)
    nw2p = _split_w(nw2)

    def kern(t_r, *refs):
        ch = refs[:nch]
        whhi_r, whlo_r, wnhi_r, wnlo_r, nb1_r, w2hi_r, w2lo_r, nb2_r, o_r \
            = refs[nch:]
        tv = t_r[...]
        coord = tv[:, 0:3]
        h = tv[:, 16:16 + f]
        tail = ch[nch - 1][...]
        tm = hid - (nch - 1) * _CW
        hn_pieces = [c_[...] for c_ in ch[:nch - 1]]
        if tm > 0:
            hn_pieces.append(tail[:, :tm])
        hn = jnp.concatenate(hn_pieces, axis=1)
        msgx = tail[:, tm:tm + 3]
        deg = tail[:, tm + 3:tm + 4]
        x_neigh = msgx / jnp.maximum(deg, 1.0)
        coord_new = coord + x_neigh
        z = _silu(_dot3(h, whhi_r[...], whlo_r[...])
                  + _dot3(hn, wnhi_r[...], wnlo_r[...])
                  + nb1_r[...])
        h_new = _dot3(z, w2hi_r[...], w2lo_r[...]) + nb2_r[...]
        h_new = jnp.maximum(h_new, 0.0)
        o_r[...] = jnp.concatenate(
            [coord_new, jnp.zeros((_BN, 13), jnp.float32), h_new], axis=1)

    full = lambda a: pl.BlockSpec(a.shape, lambda i: (0,) * a.ndim)
    return pl.pallas_call(
        kern,
        grid=(grid,),
        in_specs=[pl.BlockSpec((_BN, fp), lambda i: (i, 0))]
        + [pl.BlockSpec((_BN, _CW), lambda i: (i, 0)) for _ in range(nch)]
        + [full(whp[0]), full(whp[1]), full(wnp[0]), full(wnp[1]),
           full(nb1), full(nw2p[0]), full(nw2p[1]), full(nb2)],
        out_specs=pl.BlockSpec((_BN, fp_out), lambda i: (i, 0)),
        out_shape=jax.ShapeDtypeStruct((np_, fp_out), jnp.float32),
    )(table, *chunks, whp[0], whp[1], wnp[0], wnp[1], nb1,
      nw2p[0], nw2p[1], nb2)


def _pool_head(table, batch2, lw1p, lb1, lw2, lb2):
    np_, fp = table.shape
    _BN = _blk(np_)
    grid = np_ // _BN

    def kern(t_r, b_r, lw1_r, lb1_r, lw2_r, lb2_r, o_r, acc):
        i = pl.program_id(0)

        @pl.when(i == 0)
        def _():
            acc[...] = jnp.zeros_like(acc)

        oh = (b_r[...] == lax.broadcasted_iota(jnp.int32, (_BN, _NG), 1))
        ohf = oh.astype(jnp.float32)
        acc[...] += lax.dot_general(
            ohf, t_r[...], (((0,), (0,)), ((), ())),
            preferred_element_type=jnp.float32,
            precision=lax.Precision.HIGHEST)

        @pl.when(i == grid - 1)
        def _():
            hidden = jnp.maximum(
                jnp.dot(acc[...], lw1_r[...],
                        preferred_element_type=jnp.float32, precision=lax.Precision.HIGHEST) + lb1_r[...], 0.0)
            o_r[...] = jnp.dot(hidden, lw2_r[...],
                               preferred_element_type=jnp.float32, precision=lax.Precision.HIGHEST) + lb2_r[...]

    full = lambda a: pl.BlockSpec(a.shape, lambda i: (0,) * a.ndim)
    return pl.pallas_call(
        kern,
        grid=(grid,),
        in_specs=[pl.BlockSpec((_BN, fp), lambda i: (i, 0)),
                  pl.BlockSpec((_BN, 1), lambda i: (i, 0)),
                  full(lw1p), full(lb1), full(lw2), full(lb2)],
        out_specs=pl.BlockSpec((_NG, 1), lambda i: (0, 0)),
        out_shape=jax.ShapeDtypeStruct((_NG, 1), jnp.float32),
        scratch_shapes=[pltpu.VMEM((_NG, fp), jnp.float32)],
    )(table, batch2, lw1p, lb1, lw2, lb2)


def _ceil_to(v, m):
    return (v + m - 1) // m * m


def kernel(x, edge_index, pos, edge_attr, batch, params):
    n, f0 = x.shape
    e = edge_index.shape[1]
    ef = edge_attr.shape[1]
    hid = params["layers"][0]["ew2"].shape[1]
    nch = (hid + 16 + _CW - 1) // _CW

    ep = _ceil_to(e, _NSC * _NTILE * _K)
    np_ = _ceil_to(n, _NTILE * _K)

    pad_e = ep - e
    spread = (jnp.arange(pad_e, dtype=jnp.int32) % n).astype(jnp.int32)
    src = jnp.concatenate([edge_index[0].astype(jnp.int32), spread])
    dst = jnp.concatenate([edge_index[1].astype(jnp.int32), spread])
    ea = jnp.concatenate(
        [edge_attr, jnp.zeros((pad_e, ef), jnp.float32)], axis=0)
    batch2 = jnp.concatenate(
        [batch.astype(jnp.int32),
         jnp.full((np_ - n,), _NG, jnp.int32)])[:, None]

    table = jnp.concatenate(
        [pos, jnp.zeros((n, 13), jnp.float32), x], axis=1)
    table = jnp.concatenate(
        [table, jnp.zeros((np_ - n, 16 + f0), jnp.float32)], axis=0)
    f = f0
    fpg = _ceil_to(hid + 3, 128)
    for p in params["layers"]:
        fp = table.shape[1]

        ew1 = p["ew1"]
        zpad = jnp.zeros((16, hid), jnp.float32)
        ws = jnp.concatenate([zpad, ew1[:f]], axis=0)
        wd = jnp.concatenate([zpad, ew1[f:2 * f]], axis=0)
        wshi, wslo = _split_w(ws)
        wdhi, wdlo = _split_w(wd)
        ts, td = _project(table, wshi, wslo, wdhi, wdlo)

        gs, gd = _make_gather(ep, fpg, np_)(ts, td, src, dst)

        wr = ew1[2 * f:2 * f + 1]
        we = ew1[2 * f + 1:]
        w2hi, w2lo = _split_w(p["ew2"])
        chunks = _edge_mlp(
            gs, gd, ea, wr, we, p["eb1"][None], w2hi, w2lo,
            p["eb2"][None], p["cw1"], p["cb1"][None], p["cw2"].T,
            e, nch)

        zrows = jnp.zeros((np_ // _NTILE, _CW), jnp.float32)
        agg = _make_scatter(ep, np_, nch)(dst, zrows, *chunks)

        table = _node_update(
            table, agg, p["nw1"][:f], p["nw1"][f:], p["nb1"][None],
            p["nw2"], p["nb2"][None], nch)
        f = p["nw2"].shape[1]

    out_f = f
    lw1 = params["lw1"]
    lw1p = jnp.concatenate(
        [lw1[out_f:], jnp.zeros((13, lw1.shape[1]), jnp.float32),
         lw1[:out_f],
         jnp.zeros((table.shape[1] - 16 - out_f, lw1.shape[1]), jnp.float32)],
        axis=0)
    return _pool_head(table, batch2, lw1p, params["lb1"][None],
                      params["lw2"], params["lb2"][None])


# speedup vs baseline: 1.4920x; 1.4920x over previous
import functools

import jax
import jax.numpy as jnp
from jax import lax
from jax.experimental import pallas as pl
from jax.experimental.pallas import tpu as pltpu
from jax.experimental.pallas import tpu_sc as plsc

_NG = 64
_K = 128
_CW = 128
_NSC = 2
_NTILE = 16


def _blk(total):
    for b in (640, 512, 256, 128):
        if total % b == 0:
            return b
    return total


def _silu(v):
    return v * jax.nn.sigmoid(v)


def _split_w(w):
    hi = w.astype(jnp.bfloat16)
    lo = (w - hi.astype(jnp.float32)).astype(jnp.bfloat16)
    return hi, lo


def _dot3(v, whi, wlo):
    vhi = v.astype(jnp.bfloat16)
    vlo = (v - vhi.astype(jnp.float32)).astype(jnp.bfloat16)
    return (jnp.dot(vhi, whi, preferred_element_type=jnp.float32)
            + jnp.dot(vhi, wlo, preferred_element_type=jnp.float32)
            + jnp.dot(vlo, whi, preferred_element_type=jnp.float32))


_KG = 64


@functools.lru_cache(maxsize=None)
def _make_gather(ep, fp, np_):
    nworkers = _NSC * _NTILE
    epw = ep // nworkers
    nwin = epw // _KG
    npair = nwin // 2
    assert nwin % 2 == 0
    mesh = plsc.VectorSubcoreMesh(
        core_axis_name="c", subcore_axis_name="s",
        num_cores=_NSC, num_subcores=_NTILE)

    def body(ts, td, src, dst, gs, gd,
             idx0, idx1, rows0, rows1, gs0, gs1, os0, os1):
        wid = lax.axis_index("s") * _NSC + lax.axis_index("c")
        base0 = wid * epw
        bufs = ((idx0, rows0, gs0, os0), (idx1, rows1, gs1, os1))

        def run(table, idx_hbm, out_hbm):
            def start(w, b):
                ib, rb, gsem, _ = bufs[b]
                pltpu.sync_copy(idx_hbm.at[pl.ds(base0 + w * _KG, _KG)], ib)
                pltpu.async_copy(table.at[ib], rb, gsem)

            def gwait(b):
                ib, rb, gsem, _ = bufs[b]
                pltpu.make_async_copy(table.at[ib], rb, gsem).wait()

            def ostart(w, b):
                _, rb, _, osem = bufs[b]
                pltpu.async_copy(
                    rb, out_hbm.at[pl.ds(base0 + w * _KG, _KG)], osem)

            def owait(b):
                _, rb, _, osem = bufs[b]
                pltpu.make_async_copy(
                    rb, out_hbm.at[pl.ds(base0, _KG)], osem).wait()

            start(0, 0)

            def pair(t, carry):
                w0 = 2 * t

                @pl.when(t > 0)
                def _():
                    owait(1)
                start(w0 + 1, 1)
                gwait(0)
                ostart(w0, 0)

                @pl.when(t + 1 < npair)
                def _():
                    owait(0)
                    start(w0 + 2, 0)
                gwait(1)
                ostart(w0 + 1, 1)
                return carry
            lax.fori_loop(0, npair, pair, 0)
            owait(0)
            owait(1)

        run(ts, src, gs)
        run(td, dst, gd)

    return pl.kernel(
        body,
        out_type=(jax.ShapeDtypeStruct((ep, fp), jnp.float32),
                  jax.ShapeDtypeStruct((ep, fp), jnp.float32)),
        mesh=mesh,
        scratch_types=[pltpu.VMEM((_KG,), jnp.int32),
                       pltpu.VMEM((_KG,), jnp.int32),
                       pltpu.VMEM((_KG, fp), jnp.float32),
                       pltpu.VMEM((_KG, fp), jnp.float32),
                       pltpu.SemaphoreType.DMA,
                       pltpu.SemaphoreType.DMA,
                       pltpu.SemaphoreType.DMA,
                       pltpu.SemaphoreType.DMA],
    )


@functools.lru_cache(maxsize=None)
def _make_scatter(ep, np_, nch):
    ept = ep // _NTILE
    nwin = ept // _K
    rpt = np_ // _NTILE
    n0 = (nch + 1) // 2
    mesh = plsc.VectorSubcoreMesh(
        core_axis_name="c", subcore_axis_name="s",
        num_cores=_NSC, num_subcores=_NTILE)

    assert nwin % 2 == 0
    npair = nwin // 2

    def body(dst, zrows, *refs):
        upd = refs[:nch]
        out = refs[nch:2 * nch]
        acc, idx0, idx1, upd0, upd1, ls0, ls1 = refs[2 * nch:]
        c = lax.axis_index("c")
        s = lax.axis_index("s")
        bufs = ((idx0, upd0, ls0), (idx1, upd1, ls1))

        def process(u_hbm, o_hbm):
            ebase = s * ept

            def load(w, b):
                ib, ub, lsem = bufs[b]
                pltpu.async_copy(dst.at[pl.ds(ebase + w * _K, _K)], ib, lsem)
                pltpu.async_copy(u_hbm.at[pl.ds(ebase + w * _K, _K)], ub, lsem)

            def lwait(b):
                ib, ub, lsem = bufs[b]
                pltpu.make_async_copy(
                    dst.at[pl.ds(ebase, _K)], ib, lsem).wait()
                pltpu.make_async_copy(
                    u_hbm.at[pl.ds(ebase, _K)], ub, lsem).wait()

            pltpu.sync_copy(zrows, acc.at[pl.ds(s * rpt, rpt)])
            plsc.subcore_barrier()
            load(0, 0)

            def pair(t, carry):
                w0 = 2 * t
                lwait(0)
                load(w0 + 1, 1)
                pltpu.sync_copy(upd0, acc.at[idx0], add=True)
                lwait(1)

                @pl.when(t + 1 < npair)
                def _():
                    load(w0 + 2, 0)
                pltpu.sync_copy(upd1, acc.at[idx1], add=True)
                return carry
            lax.fori_loop(0, npair, pair, 0)
            plsc.subcore_barrier()
            pltpu.sync_copy(acc.at[pl.ds(s * rpt, rpt)],
                            o_hbm.at[pl.ds(s * rpt, rpt)])

        for j in range(n0):
            @pl.when(c == 0)
            def _(j=j):
                process(upd[j], out[j])
        for j in range(n0, nch):
            @pl.when(c == 1)
            def _(j=j):
                process(upd[j], out[j])

    return pl.kernel(
        body,
        out_type=tuple(jax.ShapeDtypeStruct((np_, _CW), jnp.float32)
                       for _ in range(nch)),
        mesh=mesh,
        scratch_types=[pltpu.VMEM_SHARED((np_, _CW), jnp.float32),
                       pltpu.VMEM((_K,), jnp.int32),
                       pltpu.VMEM((_K,), jnp.int32),
                       pltpu.VMEM((_K, _CW), jnp.float32),
                       pltpu.VMEM((_K, _CW), jnp.float32),
                       pltpu.SemaphoreType.DMA,
                       pltpu.SemaphoreType.DMA],
    )


def _project(table, wsb, wdb):
    np_, fp = table.shape
    hid = wsb.shape[1]
    fpo = _ceil_to(hid + 3, 128)
    _BN = _blk(np_)
    grid = np_ // _BN

    def kern(t_r, wsb_r, wdb_r, ts_r, td_r):
        tv = t_r[...]
        coord = tv[:, 0:3]
        tb = tv.astype(jnp.bfloat16)
        zpad = jnp.zeros((_BN, fpo - hid - 3), jnp.float32)
        for wb_r, o_r in ((wsb_r, ts_r), (wdb_r, td_r)):
            z = jnp.dot(tb, wb_r[...], preferred_element_type=jnp.float32)
            o_r[...] = jnp.concatenate([z, coord, zpad], axis=1)

    full = lambda a: pl.BlockSpec(a.shape, lambda i: (0,) * a.ndim)
    return pl.pallas_call(
        kern,
        grid=(grid,),
        in_specs=[pl.BlockSpec((_BN, fp), lambda i: (i, 0)),
                  full(wsb), full(wdb)],
        out_specs=[pl.BlockSpec((_BN, fpo), lambda i: (i, 0))] * 2,
        out_shape=[jax.ShapeDtypeStruct((np_, fpo), jnp.float32)] * 2,
    )(table, wsb, wdb)


def _edge_mlp(gs, gd, ea, wr, we, b1, w2b, b2, cw1b, cb1,
              cw2r, n_edges, nch):
    ep, fp = gs.shape
    hid = w2b.shape[1]
    ef = ea.shape[1]
    _BE = _blk(ep)
    grid = ep // _BE

    def kern(gs_r, gd_r, ea_r, wr_r, we_r, b1_r, w2b_r, b2_r,
             cw1b_r, cb1_r, cw2r_r, *outs):
        i = pl.program_id(0)
        gsv = gs_r[...]
        gdv = gd_r[...]
        xd = gsv[:, hid:hid + 3] - gdv[:, hid:hid + 3]
        radial = jnp.sum(xd * xd, axis=1, keepdims=True)
        xdn = xd / (jnp.sqrt(radial) + 1e-30)
        radial_b = radial.astype(jnp.bfloat16).astype(jnp.float32)
        z1 = (gsv[:, :hid] + gdv[:, :hid]
              + jnp.dot(ea_r[...].astype(jnp.bfloat16), we_r[...],
                        preferred_element_type=jnp.float32)
              + radial_b * wr_r[...].astype(jnp.float32) + b1_r[...])
        m1 = _silu(z1)
        m = _silu(jnp.dot(m1.astype(jnp.bfloat16), w2b_r[...],
                          preferred_element_type=jnp.float32) + b2_r[...])
        u = _silu(jnp.dot(m.astype(jnp.bfloat16), cw1b_r[...],
                          preferred_element_type=jnp.float32) + cb1_r[...])
        ub = u.astype(jnp.bfloat16).astype(jnp.float32)
        c = jnp.sum(ub * cw2r_r[...].astype(jnp.float32), axis=1,
                    keepdims=True)
        eid = i * _BE + lax.broadcasted_iota(jnp.int32, (_BE, 1), 0)
        mask = (eid < n_edges).astype(jnp.float32)
        mm = m * mask
        msgx = c * xdn * mask
        for j in range(nch - 1):
            outs[j][...] = mm[:, j * _CW:(j + 1) * _CW]
        pieces = []
        if hid > (nch - 1) * _CW:
            pieces.append(mm[:, (nch - 1) * _CW:])
        pieces += [msgx, mask]
        if nch * _CW - hid - 4 > 0:
            pieces.append(jnp.zeros((_BE, nch * _CW - hid - 4), jnp.float32))
        outs[nch - 1][...] = jnp.concatenate(pieces, axis=1)

    full = lambda a: pl.BlockSpec(a.shape, lambda i: (0,) * a.ndim)
    return pl.pallas_call(
        kern,
        grid=(grid,),
        in_specs=[
            pl.BlockSpec((_BE, fp), lambda i: (i, 0)),
            pl.BlockSpec((_BE, fp), lambda i: (i, 0)),
            pl.BlockSpec((_BE, ef), lambda i: (i, 0)),
            full(wr), full(we), full(b1),
            full(w2b), full(b2), full(cw1b), full(cb1), full(cw2r),
        ],
        out_specs=[pl.BlockSpec((_BE, _CW), lambda i: (i, 0))
                   for _ in range(nch)],
        out_shape=[jax.ShapeDtypeStruct((ep, _CW), jnp.float32)
                   for _ in range(nch)],
    )(gs, gd, ea, wr, we, b1, w2b, b2, cw1b, cb1, cw2r)


def _node_update(table, chunks, wh, wn, nb1, nw2, nb2, nch):
    np_, fp = table.shape
    f = wh.shape[0]
    hid = wn.shape[0]
    out_f = nw2.shape[1]
    fp_out = out_f + 16
    _BN = _blk(np_)
    grid = np_ // _BN
    whb = wh.astype(jnp.bfloat16)
    wnb = wn.astype(jnp.bfloat16)
    nw2b = nw2.astype(jnp.bfloat16)

    def kern(t_r, *refs):
        ch = refs[:nch]
        whb_r, wnb_r, nb1_r, nw2b_r, nb2_r, o_r = refs[nch:]
        tv = t_r[...]
        coord = tv[:, 0:3]
        h = tv[:, 16:16 + f]
        tail = ch[nch - 1][...]
        tm = hid - (nch - 1) * _CW
        hn_pieces = [c_[...] for c_ in ch[:nch - 1]]
        if tm > 0:
            hn_pieces.append(tail[:, :tm])
        hn = jnp.concatenate(hn_pieces, axis=1)
        msgx = tail[:, tm:tm + 3]
        deg = tail[:, tm + 3:tm + 4]
        x_neigh = msgx / jnp.maximum(deg, 1.0)
        coord_new = coord + x_neigh
        z = _silu(jnp.dot(h.astype(jnp.bfloat16), whb_r[...],
                          preferred_element_type=jnp.float32)
                  + jnp.dot(hn.astype(jnp.bfloat16), wnb_r[...],
                            preferred_element_type=jnp.float32)
                  + nb1_r[...])
        h_new = jnp.dot(z.astype(jnp.bfloat16), nw2b_r[...],
                        preferred_element_type=jnp.float32) + nb2_r[...]
        h_new = jnp.maximum(h_new, 0.0)
        o_r[...] = jnp.concatenate(
            [coord_new, jnp.zeros((_BN, 13), jnp.float32), h_new], axis=1)

    full = lambda a: pl.BlockSpec(a.shape, lambda i: (0,) * a.ndim)
    return pl.pallas_call(
        kern,
        grid=(grid,),
        in_specs=[pl.BlockSpec((_BN, fp), lambda i: (i, 0))]
        + [pl.BlockSpec((_BN, _CW), lambda i: (i, 0)) for _ in range(nch)]
        + [full(whb), full(wnb), full(nb1), full(nw2b), full(nb2)],
        out_specs=pl.BlockSpec((_BN, fp_out), lambda i: (i, 0)),
        out_shape=jax.ShapeDtypeStruct((np_, fp_out), jnp.float32),
    )(table, *chunks, whb, wnb, nb1, nw2b, nb2)


def _pool_head(table, batch2, lw1p, lb1, lw2, lb2):
    np_, fp = table.shape
    _BN = _blk(np_)
    grid = np_ // _BN

    def kern(t_r, b_r, lw1_r, lb1_r, lw2_r, lb2_r, o_r, acc):
        i = pl.program_id(0)

        @pl.when(i == 0)
        def _():
            acc[...] = jnp.zeros_like(acc)

        oh = (b_r[...] == lax.broadcasted_iota(jnp.int32, (_BN, _NG), 1))
        ohf = oh.astype(jnp.float32)
        acc[...] += lax.dot_general(
            ohf, t_r[...], (((0,), (0,)), ((), ())),
            preferred_element_type=jnp.float32,
            precision=lax.Precision.HIGHEST)

        @pl.when(i == grid - 1)
        def _():
            hidden = jnp.maximum(
                jnp.dot(acc[...].astype(jnp.bfloat16),
                        lw1_r[...].astype(jnp.bfloat16),
                        preferred_element_type=jnp.float32) + lb1_r[...], 0.0)
            o_r[...] = jnp.dot(hidden.astype(jnp.bfloat16),
                               lw2_r[...].astype(jnp.bfloat16),
                               preferred_element_type=jnp.float32) + lb2_r[...]

    full = lambda a: pl.BlockSpec(a.shape, lambda i: (0,) * a.ndim)
    return pl.pallas_call(
        kern,
        grid=(grid,),
        in_specs=[pl.BlockSpec((_BN, fp), lambda i: (i, 0)),
                  pl.BlockSpec((_BN, 1), lambda i: (i, 0)),
                  full(lw1p), full(lb1), full(lw2), full(lb2)],
        out_specs=pl.BlockSpec((_NG, 1), lambda i: (0, 0)),
        out_shape=jax.ShapeDtypeStruct((_NG, 1), jnp.float32),
        scratch_shapes=[pltpu.VMEM((_NG, fp), jnp.float32)],
    )(table, batch2, lw1p, lb1, lw2, lb2)


def _ceil_to(v, m):
    return (v + m - 1) // m * m


def kernel(x, edge_index, pos, edge_attr, batch, params):
    n, f0 = x.shape
    e = edge_index.shape[1]
    ef = edge_attr.shape[1]
    hid = params["layers"][0]["ew2"].shape[1]
    nch = (hid + 16 + _CW - 1) // _CW

    ep = _ceil_to(e, _NSC * _NTILE * _K)
    np_ = _ceil_to(n, _NTILE * _K)

    pad_e = ep - e
    spread = (jnp.arange(pad_e, dtype=jnp.int32) % n).astype(jnp.int32)
    src = jnp.concatenate([edge_index[0].astype(jnp.int32), spread])
    dst = jnp.concatenate([edge_index[1].astype(jnp.int32), spread])
    ea = jnp.concatenate(
        [edge_attr, jnp.zeros((pad_e, ef), jnp.float32)], axis=0)
    batch2 = jnp.concatenate(
        [batch.astype(jnp.int32),
         jnp.full((np_ - n,), _NG, jnp.int32)])[:, None]

    table = jnp.concatenate(
        [pos, jnp.zeros((n, 13), jnp.float32), x], axis=1)
    table = jnp.concatenate(
        [table, jnp.zeros((np_ - n, 16 + f0), jnp.float32)], axis=0)
    f = f0
    fpg = _ceil_to(hid + 3, 128)
    for p in params["layers"]:
        fp = table.shape[1]

        ew1 = p["ew1"]
        zpad = jnp.zeros((16, hid), jnp.float32)
        ws = jnp.concatenate([zpad, ew1[:f]], axis=0)
        wd = jnp.concatenate([zpad, ew1[f:2 * f]], axis=0)
        ts, td = _project(table, ws.astype(jnp.bfloat16),
                          wd.astype(jnp.bfloat16))

        gs, gd = _make_gather(ep, fpg, np_)(ts, td, src, dst)

        wr = ew1[2 * f:2 * f + 1].astype(jnp.bfloat16)
        we = ew1[2 * f + 1:].astype(jnp.bfloat16)
        chunks = _edge_mlp(
            gs, gd, ea, wr, we, p["eb1"][None],
            p["ew2"].astype(jnp.bfloat16), p["eb2"][None],
            p["cw1"].astype(jnp.bfloat16), p["cb1"][None],
            p["cw2"].T.astype(jnp.bfloat16), e, nch)

        zrows = jnp.zeros((np_ // _NTILE, _CW), jnp.float32)
        agg = _make_scatter(ep, np_, nch)(dst, zrows, *chunks)

        table = _node_update(
            table, agg, p["nw1"][:f], p["nw1"][f:], p["nb1"][None],
            p["nw2"], p["nb2"][None], nch)
        f = p["nw2"].shape[1]

    out_f = f
    lw1 = params["lw1"]
    lw1p = jnp.concatenate(
        [lw1[out_f:], jnp.zeros((13, lw1.shape[1]), jnp.float32),
         lw1[:out_f],
         jnp.zeros((table.shape[1] - 16 - out_f, lw1.shape[1]), jnp.float32)],
        axis=0)
    return _pool_head(table, batch2, lw1p, params["lb1"][None],
                      params["lw2"], params["lb2"][None])
